# baseline (device time: 36376 ns/iter reference)
import jax
import jax.numpy as jnp
from jax import lax
from jax.experimental import pallas as pl
from jax.experimental.pallas import tpu as pltpu

CW = 128
CQ = 8
NY = 12
NX = 11
NZ = 10


def kernel(x, W):
    t, d = x.shape
    _, v = W.shape
    qv = v // 4

    def body(x_ref, w_ref, out_ref, sums_out, sums_in,
             ysem_s, ysem_r, xsem_s, xsem_r, zsem_s, zsem_r):
        my_x = lax.axis_index("x")
        my_y = lax.axis_index("y")
        my_z = lax.axis_index("z")
        my_zp = my_z % 2
        zt = my_z - my_zp + (1 - my_zp)
        ypeer = (my_x, 1 - my_y, my_z)
        xnbr = (1 - my_x, my_y, my_z)
        ztwin = (my_x, my_y, zt)

        mb = my_y * v
        pb = (1 - my_y) * v

        def qbase(qx, qzp):
            return (2 * qx + qzp) * qv

        myq = qbase(my_x, my_zp)
        dgq = qbase(1 - my_x, 1 - my_zp)
        xq = qbase(1 - my_x, my_zp)
        zq = qbase(my_x, 1 - my_zp)

        barrier_sem = pltpu.get_barrier_semaphore()
        for nbr in (ypeer, xnbr, ztwin):
            pl.semaphore_signal(
                barrier_sem, inc=1,
                device_id=nbr, device_id_type=pl.DeviceIdType.MESH,
            )
        pl.semaphore_wait(barrier_sem, 3)

        def rcopy(col, sem_s, sem_r, k, dev):
            return pltpu.make_async_remote_copy(
                src_ref=out_ref.at[:, pl.ds(col, CW)],
                dst_ref=out_ref.at[:, pl.ds(col, CW)],
                send_sem=sem_s.at[k],
                recv_sem=sem_r.at[k],
                device_id=dev,
                device_id_type=pl.DeviceIdType.MESH,
            )

        Y = [rcopy(mb + myq + k * CW, ysem_s, ysem_r, k, ypeer)
             for k in range(CQ)]
        Y += [rcopy(mb + dgq + (k - CQ) * CW, ysem_s, ysem_r, k, ypeer)
              for k in range(CQ, NY - 1)]
        Y.append(pltpu.make_async_remote_copy(
            src_ref=sums_out,
            dst_ref=sums_in,
            send_sem=ysem_s.at[NY - 1],
            recv_sem=ysem_r.at[NY - 1],
            device_id=ypeer,
            device_id_type=pl.DeviceIdType.MESH,
        ))
        X = [rcopy(pb + myq + k * CW, xsem_s, xsem_r, k, xnbr)
             for k in range(CQ)]
        X += [rcopy(pb + zq + (3 + k - CQ) * CW, xsem_s, xsem_r, k, xnbr)
              for k in range(CQ, NX)]
        Z = [rcopy(pb + myq + k * CW, zsem_s, zsem_r, k, ztwin)
             for k in range(CQ)]
        Z += [rcopy(pb + xq + (6 + k - CQ) * CW, zsem_s, zsem_r, k, ztwin)
              for k in range(CQ, NZ)]

        xv = x_ref[...]

        def compute_quarter(qcol):
            e = jnp.exp(jnp.dot(
                xv, w_ref[:, pl.ds(qcol, qv)],
                preferred_element_type=jnp.float32,
            ))
            out_ref[:, pl.ds(mb + qcol, qv)] = e
            return jnp.sum(e, axis=1, keepdims=True)

        s = compute_quarter(myq)
        for k in range(CQ):
            Y[k].start()
        s = s + compute_quarter(dgq)
        for k in range(CQ, NY - 1):
            Y[k].start()
        s = s + compute_quarter(xq)
        s = s + compute_quarter(zq)
        sums_out[...] = jnp.broadcast_to(s, (t, CW))
        Y[NY - 1].start()

        for k in range(CQ):
            Y[k].wait_recv()
            X[k].start()
            Z[k].start()
        for k in range(CQ, NY):
            Y[k].wait_recv()

        for k in range(CQ):
            Z[k].wait_recv()
            if 3 <= k <= 5:
                X[CQ + k - 3].start()

        for k in range(CQ):
            X[k].wait_recv()
            if 6 <= k <= 7:
                Z[CQ + k - 6].start()

        for k in range(CQ, NZ):
            Z[k].wait_recv()
        for k in range(CQ, NX):
            X[k].wait_recv()

        for r in Y + X + Z:
            r.wait_send()

        s_total = s + sums_in[:, 0:1]
        out_ref[...] = out_ref[...] * (1.0 / s_total)

    return pl.pallas_call(
        body,
        out_shape=jax.ShapeDtypeStruct((t, 2 * v), jnp.float32),
        in_specs=[
            pl.BlockSpec(memory_space=pltpu.VMEM),
            pl.BlockSpec(memory_space=pltpu.VMEM),
        ],
        out_specs=pl.BlockSpec(memory_space=pltpu.VMEM),
        scratch_shapes=[
            pltpu.VMEM((t, CW), jnp.float32),
            pltpu.VMEM((t, CW), jnp.float32),
            pltpu.SemaphoreType.DMA((NY,)),
            pltpu.SemaphoreType.DMA((NY,)),
            pltpu.SemaphoreType.DMA((NX,)),
            pltpu.SemaphoreType.DMA((NX,)),
            pltpu.SemaphoreType.DMA((NZ,)),
            pltpu.SemaphoreType.DMA((NZ,)),
        ],
        compiler_params=pltpu.CompilerParams(collective_id=0),
    )(x, W)


# device time: 32535 ns/iter; 1.1181x vs baseline; 1.1181x over previous
import jax
import jax.numpy as jnp
from jax import lax
from jax.experimental import pallas as pl
from jax.experimental.pallas import tpu as pltpu

CW = 128
CQ = 8
NY = 13
NX = 10
NZ = 10
SW = 8


def kernel(x, W):
    t, d = x.shape
    _, v = W.shape
    qv = v // 4
    hq = qv // 2

    def body(x_ref, w_ref, out_ref, sums_out, sums_in,
             ysem_s, ysem_r, xsem_s, xsem_r, zsem_s, zsem_r):
        my_x = lax.axis_index("x")
        my_y = lax.axis_index("y")
        my_z = lax.axis_index("z")
        my_zp = my_z % 2
        zt = my_z - my_zp + (1 - my_zp)
        ypeer = (my_x, 1 - my_y, my_z)
        xnbr = (1 - my_x, my_y, my_z)
        ztwin = (my_x, my_y, zt)

        mb = my_y * v
        pb = (1 - my_y) * v

        def qbase(qx, qzp):
            return (2 * qx + qzp) * qv

        myq = qbase(my_x, my_zp)
        dgq = qbase(1 - my_x, 1 - my_zp)
        xq = qbase(1 - my_x, my_zp)
        zq = qbase(my_x, 1 - my_zp)

        barrier_sem = pltpu.get_barrier_semaphore()
        for nbr in (ypeer, xnbr, ztwin):
            pl.semaphore_signal(
                barrier_sem, inc=1,
                device_id=nbr, device_id_type=pl.DeviceIdType.MESH,
            )
        pl.semaphore_wait(barrier_sem, 3)

        def rcopy(col, sem_s, sem_r, k, dev):
            return pltpu.make_async_remote_copy(
                src_ref=out_ref.at[:, pl.ds(col, CW)],
                dst_ref=out_ref.at[:, pl.ds(col, CW)],
                send_sem=sem_s.at[k],
                recv_sem=sem_r.at[k],
                device_id=dev,
                device_id_type=pl.DeviceIdType.MESH,
            )

        Y = [rcopy(mb + myq + k * CW, ysem_s, ysem_r, k, ypeer)
             for k in range(CQ)]
        Y += [rcopy(mb + dgq + (k - CQ) * CW, ysem_s, ysem_r, k, ypeer)
              for k in range(CQ, NY - 1)]
        Y.append(pltpu.make_async_remote_copy(
            src_ref=sums_out,
            dst_ref=sums_in,
            send_sem=ysem_s.at[NY - 1],
            recv_sem=ysem_r.at[NY - 1],
            device_id=ypeer,
            device_id_type=pl.DeviceIdType.MESH,
        ))
        X = [rcopy(pb + myq + k * CW, xsem_s, xsem_r, k, xnbr)
             for k in range(CQ)]
        X += [rcopy(pb + zq + (4 + k - CQ) * CW, xsem_s, xsem_r, k, xnbr)
              for k in range(CQ, NX)]
        Z = [rcopy(pb + myq + k * CW, zsem_s, zsem_r, k, ztwin)
             for k in range(CQ)]
        Z += [rcopy(pb + xq + (6 + k - CQ) * CW, zsem_s, zsem_r, k, ztwin)
              for k in range(CQ, NZ)]

        xv = x_ref[...]

        def compute_cols(qcol, width):
            e = jnp.exp(jnp.dot(
                xv, w_ref[:, pl.ds(qcol, width)],
                preferred_element_type=jnp.float32,
            ))
            out_ref[:, pl.ds(mb + qcol, width)] = e
            return jnp.sum(e, axis=1, keepdims=True)

        s = compute_cols(myq, hq)
        for k in range(CQ // 2):
            Y[k].start()
        s = s + compute_cols(myq + hq, hq)
        for k in range(CQ // 2, CQ):
            Y[k].start()
        s = s + compute_cols(dgq, qv)
        for k in range(CQ, NY - 1):
            Y[k].start()
        s = s + compute_cols(xq, qv)
        s = s + compute_cols(zq, qv)
        sums_out[...] = jnp.broadcast_to(s, (t, SW))
        Y[NY - 1].start()

        for k in range(CQ):
            Y[k].wait_recv()
            X[k].start()
            Z[k].start()

        for k in (4, 5):
            Z[k].wait_recv()
            X[CQ + k - 4].start()
        for k in (6, 7):
            X[k].wait_recv()
            Z[CQ + k - 6].start()

        for k in range(CQ, NY):
            Y[k].wait_recv()
        for k in (0, 1, 2, 3, 6, 7, 8, 9):
            Z[k].wait_recv()
        for k in (0, 1, 2, 3, 4, 5, 8, 9):
            X[k].wait_recv()

        for r in Y + X + Z:
            r.wait_send()

        s_total = s + sums_in[:, 0:1]
        out_ref[...] = out_ref[...] * (1.0 / s_total)

    return pl.pallas_call(
        body,
        out_shape=jax.ShapeDtypeStruct((t, 2 * v), jnp.float32),
        in_specs=[
            pl.BlockSpec(memory_space=pltpu.VMEM),
            pl.BlockSpec(memory_space=pltpu.VMEM),
        ],
        out_specs=pl.BlockSpec(memory_space=pltpu.VMEM),
        scratch_shapes=[
            pltpu.VMEM((t, SW), jnp.float32),
            pltpu.VMEM((t, SW), jnp.float32),
            pltpu.SemaphoreType.DMA((NY,)),
            pltpu.SemaphoreType.DMA((NY,)),
            pltpu.SemaphoreType.DMA((NX,)),
            pltpu.SemaphoreType.DMA((NX,)),
            pltpu.SemaphoreType.DMA((NZ,)),
            pltpu.SemaphoreType.DMA((NZ,)),
        ],
        compiler_params=pltpu.CompilerParams(collective_id=0),
    )(x, W)
